# Initial kernel scaffold; baseline (speedup 1.0000x reference)
import jax
import jax.numpy as jnp
from jax.experimental import pallas as pl


def kernel(edge_index, edge_weight, home, away, emb, W_conv, b_conv, W1, b1, W3, b3):
    raise NotImplementedError("write your pallas kernel here")


# speedup vs baseline: 103.0436x; 103.0436x over previous
import jax
import jax.numpy as jnp
from jax import lax
from jax.experimental import pallas as pl
from jax.experimental.pallas import tpu as pltpu
from jax.experimental.pallas import tpu_sc as plsc

NC, NS, LANES = 2, 16, 16
NW = NC * NS

N_NODES = 100000
NP = 100096
SL = NP // NS
N_EDGES = 6400000
ROWS = N_EDGES // 128
WIN = 24
NWIN = 65
ROWS_MAIN = WIN * NWIN
TAIL_BASE = ROWS_MAIN * NW
TAIL_WIN = 8
TAIL_WORKERS = (ROWS - TAIL_BASE) // TAIL_WIN
WE = WIN * 128
BATCH = 16384
BROWS = BATCH // 128
BR_PER_W = BROWS // NW

_MESH = dict(core_axis_name="c", subcore_axis_name="s", num_cores=NC,
             num_subcores=NS)

_f32 = jnp.float32
_i32 = jnp.int32


def _deg_body(col_hbm, w_hbm, degp_hbm, deg_sh, colbuf, wbuf, zbuf):
  cid = lax.axis_index("c")
  sid = lax.axis_index("s")
  wid = cid * NS + sid
  sl0 = sid * SL

  def _z(i, c):
    zbuf[pl.ds(i * LANES, LANES)] = jnp.zeros((LANES,), _f32)
    return c
  lax.fori_loop(0, SL // LANES, _z, 0)
  pltpu.sync_copy(zbuf, deg_sh.at[pl.ds(sl0, SL)])
  plsc.subcore_barrier()

  base_row = wid * ROWS_MAIN

  def _win(wi, c):
    r0 = base_row + wi * WIN
    pltpu.sync_copy(col_hbm.at[pl.ds(r0, WIN)], colbuf)
    pltpu.sync_copy(w_hbm.at[pl.ds(r0 * 128, WIN * 128)], wbuf)
    for j in range(WIN):
      pltpu.sync_copy(wbuf.at[pl.ds(j * 128, 128)],
                      deg_sh.at[colbuf.at[j]], add=True)
    return c
  lax.fori_loop(0, NWIN, _win, 0)

  @pl.when(wid < TAIL_WORKERS)
  def _tail():
    r0 = TAIL_BASE + wid * TAIL_WIN
    pltpu.sync_copy(col_hbm.at[pl.ds(r0, TAIL_WIN)],
                    colbuf.at[pl.ds(0, TAIL_WIN)])
    pltpu.sync_copy(w_hbm.at[pl.ds(r0 * 128, TAIL_WIN * 128)],
                    wbuf.at[pl.ds(0, TAIL_WIN * 128)])
    for j in range(TAIL_WIN):
      pltpu.sync_copy(wbuf.at[pl.ds(j * 128, 128)],
                      deg_sh.at[colbuf.at[j]], add=True)

  plsc.subcore_barrier()
  pltpu.sync_copy(deg_sh.at[pl.ds(sl0, SL)], zbuf)
  pltpu.sync_copy(zbuf, degp_hbm.at[pl.ds(cid * NP + sl0, SL)])


def _prep_body(wc_ref, dp0_ref, dp1_ref, e0_ref, e1_ref, e2_ref,
               h0_ref, h1_ref, h2_ref, dinv_ref):
  deg = dp0_ref[...] + dp1_ref[...]
  m = deg > 0.0
  dinv = jnp.where(m, lax.rsqrt(jnp.where(m, deg, 1.0)), 0.0)
  dinv_ref[...] = dinv
  es = (e0_ref[...], e1_ref[...], e2_ref[...])
  hrefs = (h0_ref, h1_ref, h2_ref)
  for j in range(3):
    h = es[0] * wc_ref[0, j] + es[1] * wc_ref[1, j] + es[2] * wc_ref[2, j]
    hrefs[j][...] = h * dinv


def _msg_body(row_hbm, col_hbm, w_hbm, h0_hbm, h1_hbm, h2_hbm, aggp_hbm,
              hn_sh, agg_sh, rowbuf, colbuf, wbuf, m0, m1, m2, sbuf):
  cid = lax.axis_index("c")
  sid = lax.axis_index("s")
  wid = cid * NS + sid
  sl0 = sid * SL

  for p, src in enumerate((h0_hbm, h1_hbm, h2_hbm)):
    pltpu.sync_copy(src.at[pl.ds(sl0, SL)], sbuf)
    pltpu.sync_copy(sbuf, hn_sh.at[pl.ds(p * NP + sl0, SL)])

  def _z(i, c):
    sbuf[pl.ds(i * LANES, LANES)] = jnp.zeros((LANES,), _f32)
    return c
  lax.fori_loop(0, SL // LANES, _z, 0)
  for p in range(3):
    pltpu.sync_copy(sbuf, agg_sh.at[pl.ds(p * NP + sl0, SL)])

  plsc.subcore_barrier()

  base_row = wid * ROWS_MAIN
  mplanes = (m0, m1, m2)

  def _edge_window(r0, nrows):
    pltpu.sync_copy(row_hbm.at[pl.ds(r0, nrows)],
                    rowbuf.at[pl.ds(0, nrows)])
    pltpu.sync_copy(col_hbm.at[pl.ds(r0, nrows)],
                    colbuf.at[pl.ds(0, nrows)])
    pltpu.sync_copy(w_hbm.at[pl.ds(r0 * 128, nrows * 128)],
                    wbuf.at[pl.ds(0, nrows * 128)])
    for j in range(nrows):
      for p in range(3):
        pltpu.sync_copy(hn_sh.at[pl.ds(p * NP, NP)].at[rowbuf.at[j]],
                        mplanes[p].at[pl.ds(j * 128, 128)])

    def _scale(g, c):
      t = g * LANES
      w16 = wbuf[pl.ds(t, LANES)]
      m0[pl.ds(t, LANES)] = m0[pl.ds(t, LANES)] * w16
      m1[pl.ds(t, LANES)] = m1[pl.ds(t, LANES)] * w16
      m2[pl.ds(t, LANES)] = m2[pl.ds(t, LANES)] * w16
      return c
    lax.fori_loop(0, nrows * 8, _scale, 0)

    for j in range(nrows):
      for p in range(3):
        pltpu.sync_copy(mplanes[p].at[pl.ds(j * 128, 128)],
                        agg_sh.at[pl.ds(p * NP, NP)].at[colbuf.at[j]],
                        add=True)

  def _win(wi, c):
    _edge_window(base_row + wi * WIN, WIN)
    return c
  lax.fori_loop(0, NWIN, _win, 0)

  @pl.when(wid < TAIL_WORKERS)
  def _tail():
    _edge_window(TAIL_BASE + wid * TAIL_WIN, TAIL_WIN)

  plsc.subcore_barrier()
  for p in range(3):
    pltpu.sync_copy(agg_sh.at[pl.ds(p * NP + sl0, SL)], sbuf)
    pltpu.sync_copy(sbuf,
                    aggp_hbm.at[pl.ds((cid * 3 + p) * NP + sl0, SL)])


def _gather_body(aggp_hbm, dinv_hbm, home_hbm, away_hbm, bb_hbm, gh0_hbm,
                 gh1_hbm, gh2_hbm, ga0_hbm, ga1_hbm, ga2_hbm,
                 x_sh, a0, a1, dv, xbuf, bb, hidx, aidx, gbuf):
  cid = lax.axis_index("c")
  sid = lax.axis_index("s")
  wid = cid * NS + sid
  sl0 = sid * SL

  pltpu.sync_copy(dinv_hbm.at[pl.ds(sl0, SL)], dv)
  pltpu.sync_copy(bb_hbm, bb)
  pltpu.sync_copy(home_hbm, hidx)
  pltpu.sync_copy(away_hbm, aidx)

  for p in range(3):
    pltpu.sync_copy(aggp_hbm.at[pl.ds(p * NP + sl0, SL)], a0)
    pltpu.sync_copy(aggp_hbm.at[pl.ds((3 + p) * NP + sl0, SL)], a1)
    bc = bb[pl.ds(p * LANES, LANES)]

    def _node(i, c):
      o = i * LANES
      t = dv[pl.ds(o, LANES)] * (a0[pl.ds(o, LANES)] + a1[pl.ds(o, LANES)])
      t = t + bc
      xbuf[pl.ds(o, LANES)] = jnp.where(t >= 0.0, t, 0.01 * t)
      return c
    lax.fori_loop(0, SL // LANES, _node, 0)
    pltpu.sync_copy(xbuf, x_sh.at[pl.ds(p * NP + sl0, SL)])

  plsc.subcore_barrier()

  outs = ((hidx, (gh0_hbm, gh1_hbm, gh2_hbm)),
          (aidx, (ga0_hbm, ga1_hbm, ga2_hbm)))
  for r in range(BR_PER_W):
    rr = wid * BR_PER_W + r
    for idx, dsts in outs:
      for p in range(3):
        pltpu.sync_copy(x_sh.at[pl.ds(p * NP, NP)].at[idx.at[rr]], gbuf)
        pltpu.sync_copy(gbuf, dsts[p].at[pl.ds(rr * 128, 128)])


def _head_body(w1_ref, b1_ref, w3_ref, b3_ref, g0, g1, g2, g3, g4, g5,
               o0_ref, o1_ref, o2_ref):
  gs = (g0[...], g1[...], g2[...], g3[...], g4[...], g5[...])
  hs = []
  for j in range(6):
    t = b1_ref[0, j]
    for k in range(6):
      t = t + gs[k] * w1_ref[k, j]
    hs.append(jnp.where(t >= 0.0, t, 0.01 * t))
  orefs = (o0_ref, o1_ref, o2_ref)
  for j in range(3):
    t = b3_ref[0, j]
    for k in range(6):
      t = t + hs[k] * w3_ref[k, j]
    z = jnp.where(t >= 0.0, t, 0.01 * t)
    m = jnp.max(z)
    s = jnp.sum(jnp.exp(z - m))
    orefs[j][...] = z - m - jnp.log(s)


def kernel(edge_index, edge_weight, home, away, emb, W_conv, b_conv,
           W1, b1, W3, b3):
  mesh = plsc.VectorSubcoreMesh(**_MESH)

  row2d = edge_index[0].reshape(ROWS, 128).astype(_i32)
  col2d = edge_index[1].reshape(ROWS, 128).astype(_i32)
  w1d = edge_weight.reshape(N_EDGES)
  pad = NP - N_NODES
  NR = NP // 128
  eplanes = [jnp.pad(emb[:, k], (0, pad)).reshape(NR, 128) for k in range(3)]
  bb48 = jnp.repeat(b_conv, LANES)
  home2d = home.reshape(BROWS, 128).astype(_i32)
  away2d = away.reshape(BROWS, 128).astype(_i32)

  degp = pl.kernel(
      _deg_body,
      [jax.ShapeDtypeStruct((NC * NP,), _f32)],
      mesh=mesh,
      scratch_types=[
          pltpu.VMEM_SHARED((NP,), _f32),
          pltpu.VMEM((WIN, 128), _i32),
          pltpu.VMEM((WE,), _f32),
          pltpu.VMEM((SL,), _f32),
      ],
  )(col2d, w1d)[0]

  h0, h1, h2, dinv2 = pl.pallas_call(
      _prep_body,
      out_shape=[jax.ShapeDtypeStruct((NR, 128), _f32)] * 4,
      in_specs=[pl.BlockSpec(memory_space=pltpu.SMEM)]
      + [pl.BlockSpec(memory_space=pltpu.VMEM)] * 5,
  )(W_conv, degp[:NP].reshape(NR, 128), degp[NP:].reshape(NR, 128),
    *eplanes)

  aggp = pl.kernel(
      _msg_body,
      [jax.ShapeDtypeStruct((NC * 3 * NP,), _f32)],
      mesh=mesh,
      scratch_types=[
          pltpu.VMEM_SHARED((3 * NP,), _f32),
          pltpu.VMEM_SHARED((3 * NP,), _f32),
          pltpu.VMEM((WIN, 128), _i32),
          pltpu.VMEM((WIN, 128), _i32),
          pltpu.VMEM((WE,), _f32),
          pltpu.VMEM((WE,), _f32),
          pltpu.VMEM((WE,), _f32),
          pltpu.VMEM((WE,), _f32),
          pltpu.VMEM((SL,), _f32),
      ],
  )(row2d, col2d, w1d, h0.reshape(NP), h1.reshape(NP), h2.reshape(NP))[0]

  gplanes = pl.kernel(
      _gather_body,
      [jax.ShapeDtypeStruct((BATCH,), _f32)] * 6,
      mesh=mesh,
      scratch_types=[
          pltpu.VMEM_SHARED((3 * NP,), _f32),
          pltpu.VMEM((SL,), _f32),
          pltpu.VMEM((SL,), _f32),
          pltpu.VMEM((SL,), _f32),
          pltpu.VMEM((SL,), _f32),
          pltpu.VMEM((48,), _f32),
          pltpu.VMEM((BROWS, 128), _i32),
          pltpu.VMEM((BROWS, 128), _i32),
          pltpu.VMEM((128,), _f32),
      ],
  )(aggp, dinv2.reshape(NP), home2d, away2d, bb48)

  o0, o1, o2 = pl.pallas_call(
      _head_body,
      out_shape=[jax.ShapeDtypeStruct((128, 128), _f32)] * 3,
      in_specs=[pl.BlockSpec(memory_space=pltpu.SMEM)] * 4
      + [pl.BlockSpec(memory_space=pltpu.VMEM)] * 6,
  )(W1, b1.reshape(1, 6), W3, b3.reshape(1, 3),
    *[g.reshape(128, 128) for g in gplanes])

  return jnp.stack(
      [o0.reshape(BATCH), o1.reshape(BATCH), o2.reshape(BATCH)], axis=1)
